# baseline (speedup 1.0000x reference)
import functools

import jax
import jax.numpy as jnp
from jax import lax
from jax.experimental import pallas as pl
from jax.experimental.pallas import tpu as pltpu
from jax.experimental.pallas import tpu_sc as plsc

SEQ = 200
BATCH = 4096
EMBED = 64
OUT = 32
VOCAB = 1000000
VPAD = 1000064
NBLK_FULL = 7812


def _make_sc_stage(num_cores, num_subcores):
    nw = num_cores * num_subcores
    mesh = plsc.VectorSubcoreMesh(
        core_axis_name="c", subcore_axis_name="s",
        num_cores=num_cores, num_subcores=num_subcores)

    @functools.partial(
        pl.kernel,
        mesh=mesh,
        out_type=jax.ShapeDtypeStruct((VPAD, 2 * EMBED), jnp.float32),
        scratch_types=[
            pltpu.VMEM((EMBED, 128), jnp.float32),
            pltpu.VMEM((EMBED, 128), jnp.float32),
            pltpu.VMEM((128, 2 * EMBED), jnp.float32),
            pltpu.VMEM((128, 2 * EMBED), jnp.float32),
            pltpu.VMEM((EMBED, EMBED), jnp.float32),
            pltpu.SemaphoreType.DMA,
            pltpu.SemaphoreType.DMA,
            pltpu.SemaphoreType.DMA,
            pltpu.SemaphoreType.DMA,
        ],
        compiler_params=pltpu.CompilerParams(use_tc_tiling_on_sc=True,
                                             needs_layout_passes=False),
    )
    def sc_stage(tt_hbm, tail_hbm, stg_hbm, ea, eb, oa, ob, tbuf,
                 sia, sib, soa, sob):
        wid = lax.axis_index("s") * num_cores + lax.axis_index("c")
        nblk = (NBLK_FULL + 31 - wid) // 32

        def load(j, ebuf, sem):
            t = wid + 32 * j
            pltpu.async_copy(tt_hbm.at[:, pl.ds(128 * t, 128)], ebuf, sem)

        def wait_load(ebuf, sem):
            pltpu.make_async_copy(tt_hbm.at[:, pl.ds(0, 128)], ebuf,
                                  sem).wait()

        def store(j, obuf, sem):
            t = wid + 32 * j
            pltpu.async_copy(obuf, stg_hbm.at[pl.ds(128 * t, 128)], sem)

        def wait_store(obuf, sem):
            pltpu.make_async_copy(obuf, stg_hbm.at[pl.ds(0, 128)],
                                  sem).wait()

        rows_k = [lax.iota(jnp.int32, 16) + 16 * k for k in range(4)]

        def transpose(ebuf, obuf, ncols):
            def lbody(l, _):
                cl = jnp.full((16,), l, jnp.int32)
                for k in range(4):
                    v = plsc.load_gather(ebuf, [rows_k[k], cl])
                    obuf[l, pl.ds(16 * k, 16)] = v
                return 0

            lax.fori_loop(0, ncols, lbody, 0)

        @pl.when(0 < nblk)
        def _():
            load(0, ea, sia)

        @pl.when(1 < nblk)
        def _():
            load(1, eb, sib)

        def half(j, ebuf, obuf, si, so):
            @pl.when(j < nblk)
            def _():
                wait_load(ebuf, si)

                @pl.when(j >= 2)
                def _():
                    wait_store(obuf, so)

                transpose(ebuf, obuf, 128)
                store(j, obuf, so)

                @pl.when(j + 2 < nblk)
                def _():
                    load(j + 2, ebuf, si)

        def body(i, _):
            half(2 * i, ea, oa, sia, soa)
            half(2 * i + 1, eb, ob, sib, sob)
            return 0

        lax.fori_loop(0, 123, body, 0)

        @pl.when(nblk >= 1)
        def _():
            wait_store(oa, soa)

        @pl.when(nblk >= 2)
        def _():
            wait_store(ob, sob)

        @pl.when(wid == 0)
        def _():
            pltpu.sync_copy(tail_hbm, tbuf)
            transpose(tbuf, oa, 64)
            pltpu.sync_copy(oa, stg_hbm.at[pl.ds(128 * NBLK_FULL, 128)])

    return sc_stage


def _make_sc_pool(num_cores, num_subcores):
    nw = num_cores * num_subcores
    b_per_w = BATCH // nw
    mesh = plsc.VectorSubcoreMesh(
        core_axis_name="c", subcore_axis_name="s",
        num_cores=num_cores, num_subcores=num_subcores)

    @functools.partial(
        pl.kernel,
        mesh=mesh,
        out_type=jax.ShapeDtypeStruct((BATCH, EMBED), jnp.float32),
        scratch_types=[
            pltpu.VMEM((SEQ, b_per_w), jnp.int32),
            pltpu.VMEM((b_per_w, 2 * EMBED), jnp.float32),
            pltpu.VMEM((b_per_w, 2 * EMBED), jnp.float32),
            pltpu.VMEM((b_per_w, EMBED), jnp.float32),
            pltpu.SemaphoreType.DMA,
            pltpu.SemaphoreType.DMA,
        ],
        compiler_params=pltpu.CompilerParams(use_tc_tiling_on_sc=True),
    )
    def sc_pool(x_hbm, table_hbm, out_hbm, idx_v, rows_a, rows_b, acc_v,
                sem_a, sem_b):
        wid = lax.axis_index("s") * num_cores + lax.axis_index("c")
        base = wid * b_per_w

        pltpu.sync_copy(x_hbm.at[:, pl.ds(base, b_per_w)], idx_v)

        z = jnp.zeros((16,), jnp.float32)

        def zero_body(b, _):
            for j in range(EMBED // 16):
                acc_v[b, pl.ds(16 * j, 16)] = z
            return 0

        lax.fori_loop(0, b_per_w, zero_body, 0)

        def gather(s, rows, sem):
            pltpu.async_copy(table_hbm.at[idx_v.at[s]], rows, sem)

        def wait(rows, sem):
            pltpu.make_async_copy(table_hbm.at[pl.ds(0, b_per_w)], rows,
                                  sem).wait()

        def accum(rows):
            u = 4

            def body(i, _):
                b = i * u
                for k in range(u):
                    for j in range(EMBED // 16):
                        plsc.addupdate(acc_v.at[b + k, pl.ds(16 * j, 16)],
                                       rows[b + k, pl.ds(16 * j, 16)])
                return 0

            lax.fori_loop(0, b_per_w // u, body, 0)

        gather(0, rows_a, sem_a)
        gather(1, rows_b, sem_b)

        def pair(i, _):
            s0 = 2 * i
            wait(rows_a, sem_a)
            accum(rows_a)

            @pl.when(s0 + 2 < SEQ)
            def _():
                gather(s0 + 2, rows_a, sem_a)

            wait(rows_b, sem_b)
            accum(rows_b)

            @pl.when(s0 + 3 < SEQ)
            def _():
                gather(s0 + 3, rows_b, sem_b)

            return 0

        lax.fori_loop(0, SEQ // 2, pair, 0)

        pltpu.sync_copy(acc_v, out_hbm.at[pl.ds(base, b_per_w)])

    return sc_pool


def _linear_body(p_ref, w_ref, b_ref, o_ref):
    p = p_ref[...]
    w = w_ref[...]
    acc = lax.dot_general(p, w, (((1,), (1,)), ((), ())),
                          preferred_element_type=jnp.float32)
    o_ref[...] = acc * (1.0 / SEQ) + b_ref[...]


def _linear(pooled_sum, fc_w, fc_b2):
    blk = 512
    return pl.pallas_call(
        _linear_body,
        grid=(BATCH // blk,),
        in_specs=[
            pl.BlockSpec((blk, EMBED), lambda i: (i, 0)),
            pl.BlockSpec((OUT, EMBED), lambda i: (0, 0)),
            pl.BlockSpec((1, OUT), lambda i: (0, 0)),
        ],
        out_specs=pl.BlockSpec((blk, OUT), lambda i: (i, 0)),
        out_shape=jax.ShapeDtypeStruct((BATCH, OUT), jnp.float32),
    )(pooled_sum, fc_w, fc_b2)


def kernel(x, emb_table, fc_w, fc_b):
    info = plsc.get_sparse_core_info()
    sc_stage = _make_sc_stage(info.num_cores, info.num_subcores)
    sc_pool = _make_sc_pool(info.num_cores, info.num_subcores)
    staged = sc_stage(emb_table.T, emb_table[128 * NBLK_FULL:].T)
    pooled_sum = sc_pool(x.astype(jnp.int32), staged)
    return _linear(pooled_sum, fc_w, fc_b.reshape(1, OUT))


# speedup vs baseline: 3.3135x; 3.3135x over previous
import functools

import jax
import jax.numpy as jnp
from jax import lax
from jax.experimental import pallas as pl
from jax.experimental.pallas import tpu as pltpu
from jax.experimental.pallas import tpu_sc as plsc

SEQ = 200
BATCH = 4096
EMBED = 64
OUT = 32
VOCAB = 1000000
VPAD = 1000064
NBLK_FULL = 7812


def _make_sc_stage(num_cores, num_subcores):
    nw = num_cores * num_subcores
    mesh = plsc.VectorSubcoreMesh(
        core_axis_name="c", subcore_axis_name="s",
        num_cores=num_cores, num_subcores=num_subcores)

    @functools.partial(
        pl.kernel,
        mesh=mesh,
        out_type=jax.ShapeDtypeStruct((VPAD, 2 * EMBED), jnp.float32),
        scratch_types=[
            pltpu.VMEM((EMBED, 128), jnp.float32),
            pltpu.VMEM((EMBED, 128), jnp.float32),
            pltpu.VMEM((128, 2 * EMBED), jnp.float32),
            pltpu.VMEM((128, 2 * EMBED), jnp.float32),
            pltpu.VMEM((EMBED, EMBED), jnp.float32),
            pltpu.SemaphoreType.DMA,
            pltpu.SemaphoreType.DMA,
            pltpu.SemaphoreType.DMA,
            pltpu.SemaphoreType.DMA,
        ],
        compiler_params=pltpu.CompilerParams(use_tc_tiling_on_sc=True,
                                             needs_layout_passes=False),
    )
    def sc_stage(tt_hbm, tail_hbm, stg_hbm, ea, eb, oa, ob, tbuf,
                 sia, sib, soa, sob):
        wid = lax.axis_index("s") * num_cores + lax.axis_index("c")
        nblk = (NBLK_FULL + 31 - wid) // 32

        def load(j, ebuf, sem):
            t = wid + 32 * j
            pltpu.async_copy(tt_hbm.at[:, pl.ds(128 * t, 128)], ebuf, sem)

        def wait_load(ebuf, sem):
            pltpu.make_async_copy(tt_hbm.at[:, pl.ds(0, 128)], ebuf,
                                  sem).wait()

        def store(j, obuf, sem):
            t = wid + 32 * j
            pltpu.async_copy(obuf, stg_hbm.at[pl.ds(128 * t, 128)], sem)

        def wait_store(obuf, sem):
            pltpu.make_async_copy(obuf, stg_hbm.at[pl.ds(0, 128)],
                                  sem).wait()

        col_ids = [lax.iota(jnp.int32, 16) + 16 * c for c in range(8)]

        def transpose(ebuf, obuf, nchunks):
            @functools.partial(plsc.parallel_loop, 0, EMBED, unroll=4)
            def _(e):
                ce = jnp.full((16,), e, jnp.int32)
                for c in range(nchunks):
                    v = ebuf[e, pl.ds(16 * c, 16)]
                    plsc.store_scatter(obuf, [col_ids[c], ce], v)

        @pl.when(0 < nblk)
        def _():
            load(0, ea, sia)

        @pl.when(1 < nblk)
        def _():
            load(1, eb, sib)

        def half(j, ebuf, obuf, si, so):
            @pl.when(j < nblk)
            def _():
                wait_load(ebuf, si)

                @pl.when(j >= 2)
                def _():
                    wait_store(obuf, so)

                transpose(ebuf, obuf, 8)
                store(j, obuf, so)

                @pl.when(j + 2 < nblk)
                def _():
                    load(j + 2, ebuf, si)

        def body(i, _):
            half(2 * i, ea, oa, sia, soa)
            half(2 * i + 1, eb, ob, sib, sob)
            return 0

        lax.fori_loop(0, 123, body, 0)

        @pl.when(nblk >= 1)
        def _():
            wait_store(oa, soa)

        @pl.when(nblk >= 2)
        def _():
            wait_store(ob, sob)

        @pl.when(wid == 0)
        def _():
            pltpu.sync_copy(tail_hbm, tbuf)
            transpose(tbuf, oa, 4)
            pltpu.sync_copy(oa, stg_hbm.at[pl.ds(128 * NBLK_FULL, 128)])

    return sc_stage


def _make_sc_pool(num_cores, num_subcores):
    nw = num_cores * num_subcores
    b_per_w = BATCH // nw
    mesh = plsc.VectorSubcoreMesh(
        core_axis_name="c", subcore_axis_name="s",
        num_cores=num_cores, num_subcores=num_subcores)

    @functools.partial(
        pl.kernel,
        mesh=mesh,
        out_type=jax.ShapeDtypeStruct((BATCH, EMBED), jnp.float32),
        scratch_types=[
            pltpu.VMEM((SEQ, b_per_w), jnp.int32),
            pltpu.VMEM((b_per_w, 2 * EMBED), jnp.float32),
            pltpu.VMEM((b_per_w, 2 * EMBED), jnp.float32),
            pltpu.VMEM((b_per_w, EMBED), jnp.float32),
            pltpu.SemaphoreType.DMA,
            pltpu.SemaphoreType.DMA,
        ],
        compiler_params=pltpu.CompilerParams(use_tc_tiling_on_sc=True),
    )
    def sc_pool(x_hbm, table_hbm, out_hbm, idx_v, rows_a, rows_b, acc_v,
                sem_a, sem_b):
        wid = lax.axis_index("s") * num_cores + lax.axis_index("c")
        base = wid * b_per_w

        pltpu.sync_copy(x_hbm.at[:, pl.ds(base, b_per_w)], idx_v)

        z = jnp.zeros((16,), jnp.float32)

        def zero_body(b, _):
            for j in range(EMBED // 16):
                acc_v[b, pl.ds(16 * j, 16)] = z
            return 0

        lax.fori_loop(0, b_per_w, zero_body, 0)

        def gather(s, rows, sem):
            pltpu.async_copy(table_hbm.at[idx_v.at[s]], rows, sem)

        def wait(rows, sem):
            pltpu.make_async_copy(table_hbm.at[pl.ds(0, b_per_w)], rows,
                                  sem).wait()

        def accum(rows):
            u = 4

            def body(i, _):
                b = i * u
                for k in range(u):
                    for j in range(EMBED // 16):
                        plsc.addupdate(acc_v.at[b + k, pl.ds(16 * j, 16)],
                                       rows[b + k, pl.ds(16 * j, 16)])
                return 0

            lax.fori_loop(0, b_per_w // u, body, 0)

        gather(0, rows_a, sem_a)
        gather(1, rows_b, sem_b)

        def pair(i, _):
            s0 = 2 * i
            wait(rows_a, sem_a)
            accum(rows_a)

            @pl.when(s0 + 2 < SEQ)
            def _():
                gather(s0 + 2, rows_a, sem_a)

            wait(rows_b, sem_b)
            accum(rows_b)

            @pl.when(s0 + 3 < SEQ)
            def _():
                gather(s0 + 3, rows_b, sem_b)

            return 0

        lax.fori_loop(0, SEQ // 2, pair, 0)

        pltpu.sync_copy(acc_v, out_hbm.at[pl.ds(base, b_per_w)])

    return sc_pool


def _linear_body(p_ref, w_ref, b_ref, o_ref):
    p = p_ref[...]
    w = w_ref[...]
    acc = lax.dot_general(p, w, (((1,), (1,)), ((), ())),
                          preferred_element_type=jnp.float32)
    o_ref[...] = acc * (1.0 / SEQ) + b_ref[...]


def _linear(pooled_sum, fc_w, fc_b2):
    blk = 512
    return pl.pallas_call(
        _linear_body,
        grid=(BATCH // blk,),
        in_specs=[
            pl.BlockSpec((blk, EMBED), lambda i: (i, 0)),
            pl.BlockSpec((OUT, EMBED), lambda i: (0, 0)),
            pl.BlockSpec((1, OUT), lambda i: (0, 0)),
        ],
        out_specs=pl.BlockSpec((blk, OUT), lambda i: (i, 0)),
        out_shape=jax.ShapeDtypeStruct((BATCH, OUT), jnp.float32),
    )(pooled_sum, fc_w, fc_b2)


def kernel(x, emb_table, fc_w, fc_b):
    info = plsc.get_sparse_core_info()
    sc_stage = _make_sc_stage(info.num_cores, info.num_subcores)
    sc_pool = _make_sc_pool(info.num_cores, info.num_subcores)
    staged = sc_stage(emb_table.T, emb_table[128 * NBLK_FULL:].T)
    pooled_sum = sc_pool(x.astype(jnp.int32), staged)
    return _linear(pooled_sum, fc_w, fc_b.reshape(1, OUT))


# speedup vs baseline: 3.8041x; 1.1481x over previous
import functools

import jax
import jax.numpy as jnp
from jax import lax
from jax.experimental import pallas as pl
from jax.experimental.pallas import tpu as pltpu
from jax.experimental.pallas import tpu_sc as plsc

SEQ = 200
BATCH = 4096
EMBED = 64
OUT = 32
VOCAB = 1000000
VPAD = 1000064
NBLK_FULL = 7812


def _make_sc_stage(num_cores, num_subcores):
    nw = num_cores * num_subcores
    mesh = plsc.VectorSubcoreMesh(
        core_axis_name="c", subcore_axis_name="s",
        num_cores=num_cores, num_subcores=num_subcores)

    @functools.partial(
        pl.kernel,
        mesh=mesh,
        out_type=jax.ShapeDtypeStruct((VPAD, 2 * EMBED), jnp.float32),
        scratch_types=[
            pltpu.VMEM((EMBED, 128), jnp.float32),
            pltpu.VMEM((EMBED, 128), jnp.float32),
            pltpu.VMEM((EMBED, 128), jnp.float32),
            pltpu.VMEM((EMBED, 128), jnp.float32),
            pltpu.VMEM((128, 2 * EMBED), jnp.float32),
            pltpu.VMEM((128, 2 * EMBED), jnp.float32),
            pltpu.VMEM((128, 2 * EMBED), jnp.float32),
            pltpu.VMEM((128, 2 * EMBED), jnp.float32),
            pltpu.VMEM((EMBED, EMBED), jnp.float32),
            pltpu.SemaphoreType.DMA,
            pltpu.SemaphoreType.DMA,
            pltpu.SemaphoreType.DMA,
            pltpu.SemaphoreType.DMA,
            pltpu.SemaphoreType.DMA,
            pltpu.SemaphoreType.DMA,
            pltpu.SemaphoreType.DMA,
            pltpu.SemaphoreType.DMA,
        ],
        compiler_params=pltpu.CompilerParams(use_tc_tiling_on_sc=True,
                                             needs_layout_passes=False),
    )
    def sc_stage(tt_hbm, tail_hbm, stg_hbm, e0, e1, e2, e3, o0, o1, o2, o3,
                 tbuf, si0, si1, si2, si3, so0, so1, so2, so3):
        ebufs = (e0, e1, e2, e3)
        obufs = (o0, o1, o2, o3)
        sis = (si0, si1, si2, si3)
        sos = (so0, so1, so2, so3)
        wid = lax.axis_index("s") * num_cores + lax.axis_index("c")
        nblk = (NBLK_FULL + 31 - wid) // 32

        def load(j, ebuf, sem):
            t = wid + 32 * j
            pltpu.async_copy(tt_hbm.at[:, pl.ds(128 * t, 128)], ebuf, sem)

        def wait_load(ebuf, sem):
            pltpu.make_async_copy(tt_hbm.at[:, pl.ds(0, 128)], ebuf,
                                  sem).wait()

        def store(j, obuf, sem):
            t = wid + 32 * j
            pltpu.async_copy(obuf, stg_hbm.at[pl.ds(128 * t, 128)], sem)

        def wait_store(obuf, sem):
            pltpu.make_async_copy(obuf, stg_hbm.at[pl.ds(0, 128)],
                                  sem).wait()

        col_ids = [lax.iota(jnp.int32, 16) + 16 * c for c in range(8)]

        def transpose(ebuf, obuf, nchunks):
            @functools.partial(plsc.parallel_loop, 0, EMBED, unroll=4)
            def _(e):
                ce = jnp.full((16,), e, jnp.int32)
                for c in range(nchunks):
                    v = ebuf[e, pl.ds(16 * c, 16)]
                    plsc.store_scatter(obuf, [col_ids[c], ce], v)

        for q in range(4):
            @pl.when(q < nblk)
            def _(q=q):
                load(q, ebufs[q], sis[q])

        def step(j, q):
            @pl.when(j < nblk)
            def _():
                wait_load(ebufs[q], sis[q])

                @pl.when(j >= 4)
                def _():
                    wait_store(obufs[q], sos[q])

                transpose(ebufs[q], obufs[q], 8)
                store(j, obufs[q], sos[q])

                @pl.when(j + 4 < nblk)
                def _():
                    load(j + 4, ebufs[q], sis[q])

        def body(i, _):
            for q in range(4):
                step(4 * i + q, q)
            return 0

        lax.fori_loop(0, 62, body, 0)

        for q in range(4):
            @pl.when(nblk >= q + 1)
            def _(q=q):
                wait_store(obufs[q], sos[q])

        @pl.when(wid == 0)
        def _():
            pltpu.sync_copy(tail_hbm, tbuf)
            transpose(tbuf, o0, 4)
            pltpu.sync_copy(o0, stg_hbm.at[pl.ds(128 * NBLK_FULL, 128)])

    return sc_stage


def _make_sc_pool(num_cores, num_subcores):
    nw = num_cores * num_subcores
    b_per_w = BATCH // nw
    mesh = plsc.VectorSubcoreMesh(
        core_axis_name="c", subcore_axis_name="s",
        num_cores=num_cores, num_subcores=num_subcores)

    @functools.partial(
        pl.kernel,
        mesh=mesh,
        out_type=jax.ShapeDtypeStruct((BATCH, EMBED), jnp.float32),
        scratch_types=[
            pltpu.VMEM((SEQ, b_per_w), jnp.int32),
            pltpu.VMEM((b_per_w, 2 * EMBED), jnp.float32),
            pltpu.VMEM((b_per_w, 2 * EMBED), jnp.float32),
            pltpu.VMEM((b_per_w, 2 * EMBED), jnp.float32),
            pltpu.VMEM((b_per_w, 2 * EMBED), jnp.float32),
            pltpu.VMEM((b_per_w, EMBED), jnp.float32),
            pltpu.SemaphoreType.DMA,
            pltpu.SemaphoreType.DMA,
            pltpu.SemaphoreType.DMA,
            pltpu.SemaphoreType.DMA,
        ],
        compiler_params=pltpu.CompilerParams(use_tc_tiling_on_sc=True),
    )
    def sc_pool(x_hbm, table_hbm, out_hbm, idx_v, r0, r1, r2, r3, acc_v,
                s0_, s1_, s2_, s3_):
        bufs = (r0, r1, r2, r3)
        sems = (s0_, s1_, s2_, s3_)
        wid = lax.axis_index("s") * num_cores + lax.axis_index("c")
        base = wid * b_per_w

        pltpu.sync_copy(x_hbm.at[:, pl.ds(base, b_per_w)], idx_v)

        z = jnp.zeros((16,), jnp.float32)

        def zero_body(b, _):
            for j in range(EMBED // 16):
                acc_v[b, pl.ds(16 * j, 16)] = z
            return 0

        lax.fori_loop(0, b_per_w, zero_body, 0)

        def gather(s, rows, sem):
            pltpu.async_copy(table_hbm.at[idx_v.at[s]], rows, sem)

        def wait(rows, sem):
            pltpu.make_async_copy(table_hbm.at[pl.ds(0, b_per_w)], rows,
                                  sem).wait()

        def accum(rows):
            u = 4

            def body(i, _):
                b = i * u
                for k in range(u):
                    for j in range(EMBED // 16):
                        plsc.addupdate(acc_v.at[b + k, pl.ds(16 * j, 16)],
                                       rows[b + k, pl.ds(16 * j, 16)])
                return 0

            lax.fori_loop(0, b_per_w // u, body, 0)

        for q in range(4):
            gather(q, bufs[q], sems[q])

        def quad(i, _):
            s0 = 4 * i
            for q in range(4):
                wait(bufs[q], sems[q])
                accum(bufs[q])

                @pl.when(s0 + q + 4 < SEQ)
                def _():
                    gather(s0 + q + 4, bufs[q], sems[q])

            return 0

        lax.fori_loop(0, SEQ // 4, quad, 0)

        pltpu.sync_copy(acc_v, out_hbm.at[pl.ds(base, b_per_w)])

    return sc_pool


def _linear_body(p_ref, w_ref, b_ref, o_ref):
    p = p_ref[...]
    w = w_ref[...]
    acc = lax.dot_general(p, w, (((1,), (1,)), ((), ())),
                          preferred_element_type=jnp.float32)
    o_ref[...] = acc * (1.0 / SEQ) + b_ref[...]


def _linear(pooled_sum, fc_w, fc_b2):
    blk = 512
    return pl.pallas_call(
        _linear_body,
        grid=(BATCH // blk,),
        in_specs=[
            pl.BlockSpec((blk, EMBED), lambda i: (i, 0)),
            pl.BlockSpec((OUT, EMBED), lambda i: (0, 0)),
            pl.BlockSpec((1, OUT), lambda i: (0, 0)),
        ],
        out_specs=pl.BlockSpec((blk, OUT), lambda i: (i, 0)),
        out_shape=jax.ShapeDtypeStruct((BATCH, OUT), jnp.float32),
    )(pooled_sum, fc_w, fc_b2)


def kernel(x, emb_table, fc_w, fc_b):
    info = plsc.get_sparse_core_info()
    sc_stage = _make_sc_stage(info.num_cores, info.num_subcores)
    sc_pool = _make_sc_pool(info.num_cores, info.num_subcores)
    staged = sc_stage(emb_table.T, emb_table[128 * NBLK_FULL:].T)
    pooled_sum = sc_pool(x.astype(jnp.int32), staged)
    return _linear(pooled_sum, fc_w, fc_b.reshape(1, OUT))
